# Initial kernel scaffold; baseline (speedup 1.0000x reference)
import jax
import jax.numpy as jnp
from jax.experimental import pallas as pl


def kernel(batch, weights, epoch, total_epochs):
    raise NotImplementedError("write your pallas kernel here")


# speedup vs baseline: 6.5075x; 6.5075x over previous
import jax
import jax.numpy as jnp
from jax.experimental import pallas as pl
from jax.experimental.pallas import tpu as pltpu

_ROWS, _COLS = 32, 32
_LR0 = 0.5
_SIGMA0 = max(_ROWS, _COLS) / 2.0
_B, _D = 256, 512
_M = _ROWS * _COLS

_TA = 128
_TB = 256


def _bmu_body(batch_ref, w_ref, out_ref, min_ref, idx_ref):
    step = pl.program_id(0)
    w = w_ref[:]
    wn = jnp.sum(w * w, axis=1, keepdims=True)
    st = wn - 2.0 * jax.lax.dot_general(
        w, batch_ref[:], (((1,), (1,)), ((), ())),
        preferred_element_type=jnp.float32,
        precision=jax.lax.Precision.HIGHEST,
    )
    tmin = jnp.min(st, axis=0, keepdims=True)
    midx = _TA * step + jax.lax.broadcasted_iota(jnp.int32, (_TA, _B), 0)
    tidx = jnp.min(jnp.where(st == tmin, midx, _M), axis=0, keepdims=True)

    @pl.when(step == 0)
    def _():
        min_ref[0:1, :] = tmin
        idx_ref[0:1, :] = tidx

    @pl.when(step > 0)
    def _():
        better = tmin < min_ref[0:1, :]
        min_ref[0:1, :] = jnp.where(better, tmin, min_ref[0:1, :])
        idx_ref[0:1, :] = jnp.where(better, tidx, idx_ref[0:1, :])

    @pl.when(step == _M // _TA - 1)
    def _():
        out_ref[:] = jnp.broadcast_to(idx_ref[0:1, :], (8, _B))


def _update_body(scal_ref, bmu_ref, batch_ref, w_ref, out_ref):
    lr_over_b = scal_ref[0]
    neg_inv_2sig2 = scal_ref[1]
    step = pl.program_id(0)

    bmu = bmu_ref[0:1, :]
    br = (bmu // _COLS).astype(jnp.float32)
    bc = (bmu % _COLS).astype(jnp.float32)
    m2 = _TB * step + jax.lax.broadcasted_iota(jnp.int32, (_TB, _B), 0)
    mr = (m2 // _COLS).astype(jnp.float32)
    mc = (m2 % _COLS).astype(jnp.float32)
    nd2 = (mr - br) ** 2 + (mc - bc) ** 2
    h = jnp.exp(neg_inv_2sig2 * nd2)

    hsum = jnp.sum(h, axis=1, keepdims=True)
    hx = jax.lax.dot_general(
        h, batch_ref[:], (((1,), (0,)), ((), ())),
        preferred_element_type=jnp.float32,
        precision=jax.lax.Precision.HIGHEST,
    )
    w = w_ref[:]
    out_ref[:] = w + lr_over_b * (hx - hsum * w)


def kernel(batch, weights, epoch, total_epochs):
    ratio = -jnp.asarray(epoch, jnp.float32) / jnp.asarray(total_epochs, jnp.float32)
    lr = _LR0 * jnp.exp(ratio)
    sigma = _SIGMA0 * jnp.exp(ratio)
    scal = jnp.stack([lr / _B, -1.0 / (2.0 * sigma * sigma)])

    bmu = pl.pallas_call(
        _bmu_body,
        grid=(_M // _TA,),
        out_shape=jax.ShapeDtypeStruct((8, _B), jnp.int32),
        in_specs=[
            pl.BlockSpec((_B, _D), lambda i: (0, 0)),
            pl.BlockSpec((_TA, _D), lambda i: (i, 0)),
        ],
        out_specs=pl.BlockSpec((8, _B), lambda i: (0, 0)),
        scratch_shapes=[
            pltpu.VMEM((8, _B), jnp.float32),
            pltpu.VMEM((8, _B), jnp.int32),
        ],
    )(batch, weights)

    return pl.pallas_call(
        _update_body,
        grid=(_M // _TB,),
        out_shape=jax.ShapeDtypeStruct((_M, _D), jnp.float32),
        in_specs=[
            pl.BlockSpec(memory_space=pltpu.SMEM),
            pl.BlockSpec((8, _B), lambda i: (0, 0)),
            pl.BlockSpec((_B, _D), lambda i: (0, 0)),
            pl.BlockSpec((_TB, _D), lambda i: (i, 0)),
        ],
        out_specs=pl.BlockSpec((_TB, _D), lambda i: (i, 0)),
    )(scal, bmu, batch, weights)


# speedup vs baseline: 6.9839x; 1.0732x over previous
import jax
import jax.numpy as jnp
from jax.experimental import pallas as pl
from jax.experimental.pallas import tpu as pltpu

_ROWS, _COLS = 32, 32
_LR0 = 0.5
_SIGMA0 = max(_ROWS, _COLS) / 2.0
_B, _D = 256, 512
_M = _ROWS * _COLS

_T = 128
_NT = _M // _T


def _som_body(e_ref, t_ref, batch_ref, w_ref, out_ref, min_ref, idx_ref):
    i = pl.program_id(0)
    w = w_ref[:]

    @pl.when(i < _NT)
    def _bmu_phase():
        wn = jnp.sum(w * w, axis=1, keepdims=True)
        st = wn - 2.0 * jax.lax.dot_general(
            w, batch_ref[:], (((1,), (1,)), ((), ())),
            preferred_element_type=jnp.float32,
            precision=jax.lax.Precision.HIGHEST,
        )
        tmin = jnp.min(st, axis=0, keepdims=True)
        midx = _T * i + jax.lax.broadcasted_iota(jnp.int32, (_T, _B), 0)
        tidx = jnp.min(jnp.where(st == tmin, midx, _M), axis=0, keepdims=True)

        @pl.when(i == 0)
        def _():
            min_ref[0:1, :] = tmin
            idx_ref[0:1, :] = tidx

        @pl.when(i > 0)
        def _():
            better = tmin < min_ref[0:1, :]
            min_ref[0:1, :] = jnp.where(better, tmin, min_ref[0:1, :])
            idx_ref[0:1, :] = jnp.where(better, tidx, idx_ref[0:1, :])

    @pl.when(i >= _NT)
    def _update_phase():
        ratio = -(e_ref[0].astype(jnp.float32) / t_ref[0].astype(jnp.float32))
        bmu = idx_ref[0:1, :]
        br = (bmu // _COLS).astype(jnp.float32)
        bc = (bmu % _COLS).astype(jnp.float32)
        m2 = _T * (i - _NT) + jax.lax.broadcasted_iota(jnp.int32, (_T, _B), 0)
        mr = (m2 // _COLS).astype(jnp.float32)
        mc = (m2 % _COLS).astype(jnp.float32)
        nd2 = (mr - br) ** 2 + (mc - bc) ** 2
        coef = jnp.exp(jnp.full((1, _B), -2.0 * ratio)) * (-0.5 / (_SIGMA0 * _SIGMA0))
        h = jnp.exp(ratio + nd2 * coef)
        hsum = jnp.sum(h, axis=1, keepdims=True)
        hx = jax.lax.dot_general(
            h, batch_ref[:], (((1,), (0,)), ((), ())),
            preferred_element_type=jnp.float32,
        )
        out_ref[:] = w + (_LR0 / _B) * (hx - hsum * w)


def kernel(batch, weights, epoch, total_epochs):
    e = jnp.asarray(epoch, jnp.int32).reshape(1)
    t = jnp.asarray(total_epochs, jnp.int32).reshape(1)
    return pl.pallas_call(
        _som_body,
        grid=(2 * _NT,),
        out_shape=jax.ShapeDtypeStruct((_M, _D), jnp.float32),
        in_specs=[
            pl.BlockSpec(memory_space=pltpu.SMEM),
            pl.BlockSpec(memory_space=pltpu.SMEM),
            pl.BlockSpec((_B, _D), lambda i: (0, 0)),
            pl.BlockSpec((_T, _D), lambda i: (jax.lax.rem(i, _NT), 0)),
        ],
        out_specs=pl.BlockSpec(
            (_T, _D), lambda i: (jnp.where(i < _NT, 0, i - _NT), 0)
        ),
        scratch_shapes=[
            pltpu.VMEM((8, _B), jnp.float32),
            pltpu.VMEM((8, _B), jnp.int32),
        ],
    )(e, t, batch, weights)


# speedup vs baseline: 10.5357x; 1.5086x over previous
import jax
import jax.numpy as jnp
from jax.experimental import pallas as pl
from jax.experimental.pallas import tpu as pltpu

_ROWS, _COLS = 32, 32
_LR0 = 0.5
_SIGMA0 = max(_ROWS, _COLS) / 2.0
_B, _D = 256, 512
_M = _ROWS * _COLS

_T = 256
_NT = _M // _T


def _som_body(e_ref, t_ref, batch_ref, w_ref, out_ref, min_ref, idx_ref):
    i = pl.program_id(0)
    w = w_ref[:]

    @pl.when(i < _NT)
    def _bmu_phase():
        wn = jnp.sum(w * w, axis=1, keepdims=True)
        st = wn - 2.0 * jax.lax.dot_general(
            w, batch_ref[:], (((1,), (1,)), ((), ())),
            preferred_element_type=jnp.float32,
            precision=jax.lax.Precision.HIGHEST,
        )
        tmin = jnp.min(st, axis=0, keepdims=True)
        midx = _T * i + jax.lax.broadcasted_iota(jnp.int32, (_T, _B), 0)
        tidx = jnp.min(jnp.where(st == tmin, midx, _M), axis=0, keepdims=True)

        @pl.when(i == 0)
        def _():
            min_ref[0:1, :] = tmin
            idx_ref[0:1, :] = tidx

        @pl.when(i > 0)
        def _():
            better = tmin < min_ref[0:1, :]
            min_ref[0:1, :] = jnp.where(better, tmin, min_ref[0:1, :])
            idx_ref[0:1, :] = jnp.where(better, tidx, idx_ref[0:1, :])

    @pl.when(i >= _NT)
    def _update_phase():
        ratio = -(e_ref[0].astype(jnp.float32) / t_ref[0].astype(jnp.float32))
        bmu = idx_ref[0:1, :]
        br = (bmu // _COLS).astype(jnp.float32)
        bc = (bmu % _COLS).astype(jnp.float32)
        m2 = _T * (i - _NT) + jax.lax.broadcasted_iota(jnp.int32, (_T, _B), 0)
        mr = (m2 // _COLS).astype(jnp.float32)
        mc = (m2 % _COLS).astype(jnp.float32)
        nd2 = (mr - br) ** 2 + (mc - bc) ** 2
        coef = jnp.exp(jnp.full((1, _B), -2.0 * ratio)) * (-0.5 / (_SIGMA0 * _SIGMA0))
        h = jnp.exp(ratio + nd2 * coef)
        hsum = jnp.sum(h, axis=1, keepdims=True)
        hx = jax.lax.dot_general(
            h, batch_ref[:], (((1,), (0,)), ((), ())),
            preferred_element_type=jnp.float32,
        )
        out_ref[:] = w + (_LR0 / _B) * (hx - hsum * w)


def kernel(batch, weights, epoch, total_epochs):
    e = jnp.asarray(epoch, jnp.int32).reshape(1)
    t = jnp.asarray(total_epochs, jnp.int32).reshape(1)
    return pl.pallas_call(
        _som_body,
        grid=(2 * _NT,),
        out_shape=jax.ShapeDtypeStruct((_M, _D), jnp.float32),
        in_specs=[
            pl.BlockSpec(memory_space=pltpu.SMEM),
            pl.BlockSpec(memory_space=pltpu.SMEM),
            pl.BlockSpec((_B, _D), lambda i: (0, 0)),
            pl.BlockSpec((_T, _D), lambda i: (jax.lax.rem(i, _NT), 0)),
        ],
        out_specs=pl.BlockSpec(
            (_T, _D), lambda i: (jnp.where(i < _NT, 0, i - _NT), 0)
        ),
        scratch_shapes=[
            pltpu.VMEM((8, _B), jnp.float32),
            pltpu.VMEM((8, _B), jnp.int32),
        ],
    )(e, t, batch, weights)


# speedup vs baseline: 12.7527x; 1.2104x over previous
import jax
import jax.numpy as jnp
from jax.experimental import pallas as pl
from jax.experimental.pallas import tpu as pltpu

_ROWS, _COLS = 32, 32
_LR0 = 0.5
_SIGMA0 = max(_ROWS, _COLS) / 2.0
_B, _D = 256, 512
_M = _ROWS * _COLS

_T = 512
_NT = _M // _T


def _som_body(e_ref, t_ref, batch_ref, w_ref, out_ref, min_ref, idx_ref):
    i = pl.program_id(0)
    w = w_ref[:]

    @pl.when(i < _NT)
    def _bmu_phase():
        wn = jnp.sum(w * w, axis=1, keepdims=True)
        st = wn - 2.0 * jax.lax.dot_general(
            w, batch_ref[:], (((1,), (1,)), ((), ())),
            preferred_element_type=jnp.float32,
            precision=jax.lax.Precision.HIGHEST,
        )
        tmin = jnp.min(st, axis=0, keepdims=True)
        midx = _T * i + jax.lax.broadcasted_iota(jnp.int32, (_T, _B), 0)
        tidx = jnp.min(jnp.where(st == tmin, midx, _M), axis=0, keepdims=True)

        @pl.when(i == 0)
        def _():
            min_ref[0:1, :] = tmin
            idx_ref[0:1, :] = tidx

        @pl.when(i > 0)
        def _():
            better = tmin < min_ref[0:1, :]
            min_ref[0:1, :] = jnp.where(better, tmin, min_ref[0:1, :])
            idx_ref[0:1, :] = jnp.where(better, tidx, idx_ref[0:1, :])

    @pl.when(i >= _NT)
    def _update_phase():
        ratio = -(e_ref[0].astype(jnp.float32) / t_ref[0].astype(jnp.float32))
        bmu = idx_ref[0:1, :]
        br = (bmu // _COLS).astype(jnp.float32)
        bc = (bmu % _COLS).astype(jnp.float32)
        m2 = _T * (i - _NT) + jax.lax.broadcasted_iota(jnp.int32, (_T, _B), 0)
        mr = (m2 // _COLS).astype(jnp.float32)
        mc = (m2 % _COLS).astype(jnp.float32)
        nd2 = (mr - br) ** 2 + (mc - bc) ** 2
        coef = jnp.exp(jnp.full((1, _B), -2.0 * ratio)) * (-0.5 / (_SIGMA0 * _SIGMA0))
        h = jnp.exp(ratio + nd2 * coef)
        hsum = jnp.sum(h, axis=1, keepdims=True)
        hx = jax.lax.dot_general(
            h, batch_ref[:], (((1,), (0,)), ((), ())),
            preferred_element_type=jnp.float32,
        )
        out_ref[:] = w + (_LR0 / _B) * (hx - hsum * w)


def kernel(batch, weights, epoch, total_epochs):
    e = jnp.asarray(epoch, jnp.int32).reshape(1)
    t = jnp.asarray(total_epochs, jnp.int32).reshape(1)
    return pl.pallas_call(
        _som_body,
        grid=(2 * _NT,),
        out_shape=jax.ShapeDtypeStruct((_M, _D), jnp.float32),
        in_specs=[
            pl.BlockSpec(memory_space=pltpu.SMEM),
            pl.BlockSpec(memory_space=pltpu.SMEM),
            pl.BlockSpec((_B, _D), lambda i: (0, 0)),
            pl.BlockSpec((_T, _D), lambda i: (jax.lax.rem(i, _NT), 0)),
        ],
        out_specs=pl.BlockSpec(
            (_T, _D), lambda i: (jnp.where(i < _NT, 0, i - _NT), 0)
        ),
        scratch_shapes=[
            pltpu.VMEM((8, _B), jnp.float32),
            pltpu.VMEM((8, _B), jnp.int32),
        ],
    )(e, t, batch, weights)


# speedup vs baseline: 13.4887x; 1.0577x over previous
import jax
import jax.numpy as jnp
from jax.experimental import pallas as pl
from jax.experimental.pallas import tpu as pltpu

_ROWS, _COLS = 32, 32
_LR0 = 0.5
_SIGMA0 = max(_ROWS, _COLS) / 2.0
_B, _D = 256, 512
_M = _ROWS * _COLS

_T = 1024
_NT = _M // _T


def _som_body(e_ref, t_ref, batch_ref, w_ref, out_ref, min_ref, idx_ref):
    i = pl.program_id(0)
    w = w_ref[:]

    @pl.when(i < _NT)
    def _bmu_phase():
        wn = jnp.sum(w * w, axis=1, keepdims=True)
        st = wn - 2.0 * jax.lax.dot_general(
            w, batch_ref[:], (((1,), (1,)), ((), ())),
            preferred_element_type=jnp.float32,
            precision=jax.lax.Precision.HIGHEST,
        )
        tmin = jnp.min(st, axis=0, keepdims=True)
        midx = _T * i + jax.lax.broadcasted_iota(jnp.int32, (_T, _B), 0)
        tidx = jnp.min(jnp.where(st == tmin, midx, _M), axis=0, keepdims=True)

        @pl.when(i == 0)
        def _():
            min_ref[0:1, :] = tmin
            idx_ref[0:1, :] = tidx

        @pl.when(i > 0)
        def _():
            better = tmin < min_ref[0:1, :]
            min_ref[0:1, :] = jnp.where(better, tmin, min_ref[0:1, :])
            idx_ref[0:1, :] = jnp.where(better, tidx, idx_ref[0:1, :])

    @pl.when(i >= _NT)
    def _update_phase():
        ratio = -(e_ref[0].astype(jnp.float32) / t_ref[0].astype(jnp.float32))
        bmu = idx_ref[0:1, :]
        br = (bmu // _COLS).astype(jnp.float32)
        bc = (bmu % _COLS).astype(jnp.float32)
        m2 = _T * (i - _NT) + jax.lax.broadcasted_iota(jnp.int32, (_T, _B), 0)
        mr = (m2 // _COLS).astype(jnp.float32)
        mc = (m2 % _COLS).astype(jnp.float32)
        nd2 = (mr - br) ** 2 + (mc - bc) ** 2
        coef = jnp.exp(jnp.full((1, _B), -2.0 * ratio)) * (-0.5 / (_SIGMA0 * _SIGMA0))
        h = jnp.exp(ratio + nd2 * coef)
        hsum = jnp.sum(h, axis=1, keepdims=True)
        hx = jax.lax.dot_general(
            h, batch_ref[:], (((1,), (0,)), ((), ())),
            preferred_element_type=jnp.float32,
        )
        out_ref[:] = w + (_LR0 / _B) * (hx - hsum * w)


def kernel(batch, weights, epoch, total_epochs):
    e = jnp.asarray(epoch, jnp.int32).reshape(1)
    t = jnp.asarray(total_epochs, jnp.int32).reshape(1)
    return pl.pallas_call(
        _som_body,
        grid=(2 * _NT,),
        out_shape=jax.ShapeDtypeStruct((_M, _D), jnp.float32),
        in_specs=[
            pl.BlockSpec(memory_space=pltpu.SMEM),
            pl.BlockSpec(memory_space=pltpu.SMEM),
            pl.BlockSpec((_B, _D), lambda i: (0, 0)),
            pl.BlockSpec((_T, _D), lambda i: (jax.lax.rem(i, _NT), 0)),
        ],
        out_specs=pl.BlockSpec(
            (_T, _D), lambda i: (jnp.where(i < _NT, 0, i - _NT), 0)
        ),
        scratch_shapes=[
            pltpu.VMEM((8, _B), jnp.float32),
            pltpu.VMEM((8, _B), jnp.int32),
        ],
    )(e, t, batch, weights)
